# Initial kernel scaffold; baseline (speedup 1.0000x reference)
import jax
import jax.numpy as jnp
from jax.experimental import pallas as pl


def kernel(x, edge_index, initial_coords, W_node, b_node, W_coord, b_coord, W_q, b_q, W_k, b_k, W_v, b_v, W_d1, b_d1, W_d2, b_d2, W_force, b_force):
    raise NotImplementedError("write your pallas kernel here")


# speedup vs baseline: 3.1993x; 3.1993x over previous
import functools

import jax
import jax.numpy as jnp
from jax import lax
from jax.experimental import pallas as pl
from jax.experimental.pallas import tpu as pltpu
from jax.experimental.pallas import tpu_sc as plsc

N = 10000
D = 128
H = 128
TOP_K = 32
N_PAD = 10240

_PREC = jax.lax.Precision.HIGHEST


def _dotf(a, b):
    return jax.lax.dot_general(a, b, (((1,), (0,)), ((), ())),
                               precision=_PREC,
                               preferred_element_type=jnp.float32)


def _proj_body(x_ref, c_ref, wn_ref, bn_ref, wc_ref, bc_ref,
               wqa_ref, wqb_ref, bq_ref, wka_ref, wkb_ref, bk_ref,
               wva_ref, wvb_ref, bv_ref, wf_ref, bf_ref,
               q_ref, k_ref, vf_ref):
    h_node = _dotf(x_ref[...], wn_ref[...]) + bn_ref[...]
    h_coord = _dotf(c_ref[...], wc_ref[...]) + bc_ref[...]
    q = _dotf(h_node, wqa_ref[...]) + _dotf(h_coord, wqb_ref[...]) + bq_ref[...]
    k = _dotf(h_node, wka_ref[...]) + _dotf(h_coord, wkb_ref[...]) + bk_ref[...]
    v = _dotf(h_node, wva_ref[...]) + _dotf(h_coord, wvb_ref[...]) + bv_ref[...]
    q_ref[...] = q
    k_ref[...] = k
    vf_ref[...] = _dotf(v, wf_ref[...]) + bf_ref[...]


def _projections(x_pad, c_pad, W_node, b_node, W_coord, b_coord,
                 W_q, b_q, W_k, b_k, W_v, b_v, Wf_pad, bf_pad):
    BLK = 2048
    grid = (N_PAD // BLK,)
    row_spec2 = lambda d: pl.BlockSpec((BLK, d), lambda i: (i, 0))
    full = lambda a: pl.BlockSpec(a.shape, lambda i: (0,) * a.ndim)
    args = (x_pad, c_pad, W_node, b_node.reshape(1, H),
            W_coord, b_coord.reshape(1, H // 4),
            W_q[:H], W_q[H:], b_q.reshape(1, H),
            W_k[:H], W_k[H:], b_k.reshape(1, H),
            W_v[:H], W_v[H:], b_v.reshape(1, H),
            Wf_pad, bf_pad)
    in_specs = [row_spec2(D), row_spec2(2)] + [full(a) for a in args[2:]]
    return pl.pallas_call(
        _proj_body,
        grid=grid,
        in_specs=in_specs,
        out_specs=[row_spec2(H), row_spec2(H), row_spec2(128)],
        out_shape=[
            jax.ShapeDtypeStruct((N_PAD, H), jnp.float32),
            jax.ShapeDtypeStruct((N_PAD, H), jnp.float32),
            jax.ShapeDtypeStruct((N_PAD, 128), jnp.float32),
        ],
    )(*args)


def _topk_body(xc_ref, yc_ref, xr_ref, yr_ref, q_ref, k_ref, idx_ref,
               w_ref):
    R = xc_ref.shape[0]
    xi = xc_ref[...]
    yi = yc_ref[...]
    xj = xr_ref[...]
    yj = yr_ref[...]
    sq_r = xi * xi + yi * yi
    sq_c = xj * xj + yj * yj
    xbi = xi.astype(jnp.bfloat16).astype(jnp.float32)
    ybi = yi.astype(jnp.bfloat16).astype(jnp.float32)
    xbj = xj.astype(jnp.bfloat16).astype(jnp.float32)
    ybj = yj.astype(jnp.bfloat16).astype(jnp.float32)
    dot = xbi * xbj + ybi * ybj
    d2 = (sq_r + sq_c) - 2.0 * dot
    d2 = jnp.maximum(d2, 0.0)
    dist = jnp.where(d2 > 1e-12, jnp.sqrt(jnp.where(d2 > 1e-12, d2, 1e-12)),
                     0.0)
    qk = jax.lax.dot_general(q_ref[...], k_ref[...], (((1,), (1,)), ((), ())),
                             precision=_PREC,
                             preferred_element_type=jnp.float32)
    a_full = qk * (1.0 / (H ** 0.5)) + 1.0 / (dist + 1e-6)

    col = lax.broadcasted_iota(jnp.int32, (R, N), 1)
    lane_k = lax.broadcasted_iota(jnp.int32, (1, TOP_K), 1)
    work = dist
    acc_idx = jnp.zeros((R, TOP_K), jnp.int32)
    acc_a = jnp.zeros((R, TOP_K), jnp.float32)
    for k in range(TOP_K):
        m = jnp.min(work, axis=1, keepdims=True)
        hit = work == m
        idxk = jnp.min(jnp.where(hit, col, jnp.int32(2 ** 30)), axis=1,
                       keepdims=True)
        onehot = col == idxk
        a_k = jnp.sum(jnp.where(onehot, a_full, 0.0), axis=1, keepdims=True)
        sel = lane_k == k
        acc_idx = jnp.where(sel, idxk, acc_idx)
        acc_a = jnp.where(sel, a_k, acc_a)
        work = jnp.where(onehot, jnp.float32(jnp.inf), work)

    amax = jnp.max(acc_a, axis=1, keepdims=True)
    e = jnp.exp(acc_a - amax)
    w = e / jnp.sum(e, axis=1, keepdims=True)
    idx_ref[...] = acc_idx
    w_ref[...] = w


def _topk_attn(xc, yc, xr, yr, Q, K):
    R = 128
    grid = (N_PAD // R,)
    return pl.pallas_call(
        _topk_body,
        grid=grid,
        in_specs=[
            pl.BlockSpec((R, 1), lambda i: (i, 0)),
            pl.BlockSpec((R, 1), lambda i: (i, 0)),
            pl.BlockSpec((1, N), lambda i: (0, 0)),
            pl.BlockSpec((1, N), lambda i: (0, 0)),
            pl.BlockSpec((R, H), lambda i: (i, 0)),
            pl.BlockSpec((N, H), lambda i: (0, 0)),
        ],
        out_specs=[
            pl.BlockSpec((R, TOP_K), lambda i: (i, 0)),
            pl.BlockSpec((R, TOP_K), lambda i: (i, 0)),
        ],
        out_shape=[
            jax.ShapeDtypeStruct((N_PAD, TOP_K), jnp.int32),
            jax.ShapeDtypeStruct((N_PAD, TOP_K), jnp.float32),
        ],
    )(xc, yc, xr, yr, Q, K)


def _gather_sum(vf, idx_flat, w_flat):
    info = plsc.get_sparse_core_info()
    NC, NS = info.num_cores, info.num_subcores
    NW = NC * NS
    b_per_w = N_PAD // NW
    per_w = b_per_w * TOP_K
    CHUNK_NODES = 4
    CHUNK = CHUNK_NODES * TOP_K
    n_chunks = b_per_w // CHUNK_NODES
    mesh = plsc.VectorSubcoreMesh(core_axis_name="c", subcore_axis_name="s")

    @functools.partial(
        pl.kernel, mesh=mesh,
        compiler_params=pltpu.CompilerParams(needs_layout_passes=False),
        out_type=jax.ShapeDtypeStruct((N_PAD, 16), jnp.float32),
        scratch_types=[
            pltpu.VMEM((per_w,), jnp.int32),
            pltpu.VMEM((per_w,), jnp.float32),
            pltpu.VMEM((CHUNK, 128), jnp.float32),
            pltpu.VMEM((b_per_w, 16), jnp.float32),
            pltpu.SemaphoreType.DMA,
        ],
    )
    def sc_kernel(vf_hbm, idx_hbm, w_hbm, out_hbm, idx_v, w_v, rows_v,
                  out_v, sem):
        wid = lax.axis_index("s") * NC + lax.axis_index("c")
        base = wid * per_w
        pltpu.sync_copy(idx_hbm.at[pl.ds(base, per_w)], idx_v)
        pltpu.sync_copy(w_hbm.at[pl.ds(base, per_w)], w_v)

        def chunk_body(c, carry):
            off = c * CHUNK
            pltpu.async_copy(vf_hbm.at[idx_v.at[pl.ds(off, CHUNK)]], rows_v,
                             sem).wait()
            for t in range(CHUNK_NODES):
                acc = jnp.zeros((16,), jnp.float32)
                for j in range(TOP_K):
                    p = t * TOP_K + j
                    wj = plsc.load_gather(
                        w_v, [jnp.full((16,), off + p, jnp.int32)])
                    acc = acc + wj * rows_v[p, 0:16]
                out_v[c * CHUNK_NODES + t, :] = acc
            return carry

        lax.fori_loop(0, n_chunks, chunk_body, 0)
        pltpu.sync_copy(out_v, out_hbm.at[pl.ds(wid * b_per_w, b_per_w)])

    return sc_kernel(vf, idx_flat, w_flat)


def kernel(x, edge_index, initial_coords, W_node, b_node, W_coord, b_coord,
           W_q, b_q, W_k, b_k, W_v, b_v, W_d1, b_d1, W_d2, b_d2,
           W_force, b_force):
    del edge_index, W_d1, b_d1, W_d2, b_d2
    x = x.astype(jnp.float32)
    coords = initial_coords.astype(jnp.float32)
    x_pad = jnp.pad(x, ((0, N_PAD - N), (0, 0)))
    c_pad = jnp.pad(coords, ((0, N_PAD - N), (0, 0)))
    Wf_pad = jnp.pad(W_force, ((0, 0), (0, 126)))
    bf_pad = jnp.pad(b_force, (0, 126)).reshape(1, 128)

    Q, K, VF = _projections(x_pad, c_pad, W_node, b_node, W_coord, b_coord,
                            W_q, b_q, W_k, b_k, W_v, b_v, Wf_pad, bf_pad)
    xc = c_pad[:, 0:1]
    yc = c_pad[:, 1:2]
    xr = coords[:, 0].reshape(1, N)
    yr = coords[:, 1].reshape(1, N)
    idx, w = _topk_attn(xc, yc, xr, yr, Q, K[:N])
    out = _gather_sum(VF, idx.reshape(-1), w.reshape(-1))
    return out[:N, :2]


# speedup vs baseline: 4.1086x; 1.2842x over previous
import functools

import jax
import jax.numpy as jnp
from jax import lax
from jax.experimental import pallas as pl
from jax.experimental.pallas import tpu as pltpu
from jax.experimental.pallas import tpu_sc as plsc

N = 10000
D = 128
H = 128
TOP_K = 32
N_PAD = 10240

_PREC = jax.lax.Precision.HIGHEST


def _dotf(a, b):
    return jax.lax.dot_general(a, b, (((1,), (0,)), ((), ())),
                               precision=_PREC,
                               preferred_element_type=jnp.float32)


def _proj_body(x_ref, c_ref, wn_ref, bn_ref, wc_ref, bc_ref,
               wqa_ref, wqb_ref, bq_ref, wka_ref, wkb_ref, bk_ref,
               wva_ref, wvb_ref, bv_ref, wf_ref, bf_ref,
               q_ref, k_ref, vf_ref):
    h_node = _dotf(x_ref[...], wn_ref[...]) + bn_ref[...]
    h_coord = _dotf(c_ref[...], wc_ref[...]) + bc_ref[...]
    q = _dotf(h_node, wqa_ref[...]) + _dotf(h_coord, wqb_ref[...]) + bq_ref[...]
    k = _dotf(h_node, wka_ref[...]) + _dotf(h_coord, wkb_ref[...]) + bk_ref[...]
    v = _dotf(h_node, wva_ref[...]) + _dotf(h_coord, wvb_ref[...]) + bv_ref[...]
    q_ref[...] = q
    k_ref[...] = k
    vf_ref[...] = _dotf(v, wf_ref[...]) + bf_ref[...]


def _projections(x_pad, c_pad, W_node, b_node, W_coord, b_coord,
                 W_q, b_q, W_k, b_k, W_v, b_v, Wf_pad, bf_pad):
    BLK = 2048
    grid = (N_PAD // BLK,)
    row_spec2 = lambda d: pl.BlockSpec((BLK, d), lambda i: (i, 0))
    full = lambda a: pl.BlockSpec(a.shape, lambda i: (0,) * a.ndim)
    args = (x_pad, c_pad, W_node, b_node.reshape(1, H),
            W_coord, b_coord.reshape(1, H // 4),
            W_q[:H], W_q[H:], b_q.reshape(1, H),
            W_k[:H], W_k[H:], b_k.reshape(1, H),
            W_v[:H], W_v[H:], b_v.reshape(1, H),
            Wf_pad, bf_pad)
    in_specs = [row_spec2(D), row_spec2(2)] + [full(a) for a in args[2:]]
    return pl.pallas_call(
        _proj_body,
        grid=grid,
        in_specs=in_specs,
        out_specs=[row_spec2(H), row_spec2(H), row_spec2(128)],
        out_shape=[
            jax.ShapeDtypeStruct((N_PAD, H), jnp.float32),
            jax.ShapeDtypeStruct((N_PAD, H), jnp.float32),
            jax.ShapeDtypeStruct((N_PAD, 128), jnp.float32),
        ],
    )(*args)


def _topk_body(xc_ref, yc_ref, xr_ref, yr_ref, q_ref, k_ref, idx_ref,
               w_ref):
    R = xc_ref.shape[0]
    xi = xc_ref[...]
    yi = yc_ref[...]
    xj = xr_ref[...]
    yj = yr_ref[...]
    sq_r = xi * xi + yi * yi
    sq_c = xj * xj + yj * yj
    xbi = xi.astype(jnp.bfloat16).astype(jnp.float32)
    ybi = yi.astype(jnp.bfloat16).astype(jnp.float32)
    xbj = xj.astype(jnp.bfloat16).astype(jnp.float32)
    ybj = yj.astype(jnp.bfloat16).astype(jnp.float32)
    dot = xbi * xbj + ybi * ybj
    d2 = (sq_r + sq_c) - 2.0 * dot
    d2 = jnp.maximum(d2, 0.0)
    dist = jnp.where(d2 > 1e-12, jnp.sqrt(jnp.where(d2 > 1e-12, d2, 1e-12)),
                     0.0)
    qk = jax.lax.dot_general(q_ref[...], k_ref[...], (((1,), (1,)), ((), ())),
                             precision=_PREC,
                             preferred_element_type=jnp.float32)
    a_full = qk * (1.0 / (H ** 0.5)) + 1.0 / (dist + 1e-6)

    col = lax.broadcasted_iota(jnp.int32, (R, N), 1)
    lane_k = lax.broadcasted_iota(jnp.int32, (1, TOP_K), 1)
    bits = lax.bitcast_convert_type(dist, jnp.int32)
    key = (bits & jnp.int32(-16384)) | col
    acc_idx = jnp.zeros((R, TOP_K), jnp.int32)
    acc_a = jnp.zeros((R, TOP_K), jnp.float32)
    for k in range(TOP_K):
        mk = jnp.min(key, axis=1, keepdims=True)
        idxk = mk & jnp.int32(16383)
        onehot = col == idxk
        a_k = jnp.sum(jnp.where(onehot, a_full, 0.0), axis=1, keepdims=True)
        sel = lane_k == k
        acc_idx = jnp.where(sel, idxk, acc_idx)
        acc_a = jnp.where(sel, a_k, acc_a)
        key = jnp.where(onehot, jnp.int32(2147483647), key)

    amax = jnp.max(acc_a, axis=1, keepdims=True)
    e = jnp.exp(acc_a - amax)
    w = e / jnp.sum(e, axis=1, keepdims=True)
    idx_ref[...] = acc_idx
    w_ref[...] = w


def _topk_attn(xc, yc, xr, yr, Q, K):
    R = 128
    grid = (N_PAD // R,)
    return pl.pallas_call(
        _topk_body,
        grid=grid,
        in_specs=[
            pl.BlockSpec((R, 1), lambda i: (i, 0)),
            pl.BlockSpec((R, 1), lambda i: (i, 0)),
            pl.BlockSpec((1, N), lambda i: (0, 0)),
            pl.BlockSpec((1, N), lambda i: (0, 0)),
            pl.BlockSpec((R, H), lambda i: (i, 0)),
            pl.BlockSpec((N, H), lambda i: (0, 0)),
        ],
        out_specs=[
            pl.BlockSpec((R, TOP_K), lambda i: (i, 0)),
            pl.BlockSpec((R, TOP_K), lambda i: (i, 0)),
        ],
        out_shape=[
            jax.ShapeDtypeStruct((N_PAD, TOP_K), jnp.int32),
            jax.ShapeDtypeStruct((N_PAD, TOP_K), jnp.float32),
        ],
    )(xc, yc, xr, yr, Q, K)


def _gather_sum(vf, idx_flat, w_flat):
    info = plsc.get_sparse_core_info()
    NC, NS = info.num_cores, info.num_subcores
    NW = NC * NS
    b_per_w = N_PAD // NW
    per_w = b_per_w * TOP_K
    CHUNK_NODES = 4
    CHUNK = CHUNK_NODES * TOP_K
    n_chunks = b_per_w // CHUNK_NODES
    mesh = plsc.VectorSubcoreMesh(core_axis_name="c", subcore_axis_name="s")

    @functools.partial(
        pl.kernel, mesh=mesh,
        compiler_params=pltpu.CompilerParams(needs_layout_passes=False),
        out_type=jax.ShapeDtypeStruct((N_PAD, 16), jnp.float32),
        scratch_types=[
            pltpu.VMEM((per_w,), jnp.int32),
            pltpu.VMEM((per_w,), jnp.float32),
            pltpu.VMEM((CHUNK, 128), jnp.float32),
            pltpu.VMEM((b_per_w, 16), jnp.float32),
            pltpu.SemaphoreType.DMA,
        ],
    )
    def sc_kernel(vf_hbm, idx_hbm, w_hbm, out_hbm, idx_v, w_v, rows_v,
                  out_v, sem):
        wid = lax.axis_index("s") * NC + lax.axis_index("c")
        base = wid * per_w
        pltpu.sync_copy(idx_hbm.at[pl.ds(base, per_w)], idx_v)
        pltpu.sync_copy(w_hbm.at[pl.ds(base, per_w)], w_v)

        def chunk_body(c, carry):
            off = c * CHUNK
            pltpu.async_copy(vf_hbm.at[idx_v.at[pl.ds(off, CHUNK)]], rows_v,
                             sem).wait()
            for t in range(CHUNK_NODES):
                acc = jnp.zeros((16,), jnp.float32)
                for j in range(TOP_K):
                    p = t * TOP_K + j
                    wj = plsc.load_gather(
                        w_v, [jnp.full((16,), off + p, jnp.int32)])
                    acc = acc + wj * rows_v[p, 0:16]
                out_v[c * CHUNK_NODES + t, :] = acc
            return carry

        lax.fori_loop(0, n_chunks, chunk_body, 0)
        pltpu.sync_copy(out_v, out_hbm.at[pl.ds(wid * b_per_w, b_per_w)])

    return sc_kernel(vf, idx_flat, w_flat)


def kernel(x, edge_index, initial_coords, W_node, b_node, W_coord, b_coord,
           W_q, b_q, W_k, b_k, W_v, b_v, W_d1, b_d1, W_d2, b_d2,
           W_force, b_force):
    del edge_index, W_d1, b_d1, W_d2, b_d2
    x = x.astype(jnp.float32)
    coords = initial_coords.astype(jnp.float32)
    x_pad = jnp.pad(x, ((0, N_PAD - N), (0, 0)))
    c_pad = jnp.pad(coords, ((0, N_PAD - N), (0, 0)))
    Wf_pad = jnp.pad(W_force, ((0, 0), (0, 126)))
    bf_pad = jnp.pad(b_force, (0, 126)).reshape(1, 128)

    Q, K, VF = _projections(x_pad, c_pad, W_node, b_node, W_coord, b_coord,
                            W_q, b_q, W_k, b_k, W_v, b_v, Wf_pad, bf_pad)
    xc = c_pad[:, 0:1]
    yc = c_pad[:, 1:2]
    xr = coords[:, 0].reshape(1, N)
    yr = coords[:, 1].reshape(1, N)
    idx, w = _topk_attn(xc, yc, xr, yr, Q, K[:N])
    out = _gather_sum(VF, idx.reshape(-1), w.reshape(-1))
    return out[:N, :2]
